# baseline (speedup 1.0000x reference)
import functools

import jax
import jax.numpy as jnp
from jax import lax
from jax.experimental import pallas as pl
from jax.experimental.pallas import tpu as pltpu
from jax.experimental.pallas import tpu_sc as plsc

NUM_EMB = 1000000
DIM = 32
B_TOTAL = 16384 * 26
NUM_WORKERS = 32
BPW = B_TOTAL // NUM_WORKERS
CHUNK = 512
NCHUNK = BPW // CHUNK
NBUF = 4

NTILE = NUM_EMB // 128
TAIL = NUM_EMB - NTILE * 128
NQUAD = NUM_EMB // 4

_mesh = plsc.VectorSubcoreMesh(core_axis_name="c", subcore_axis_name="s")


def _transpose_slab(slab, block, ncols):
    iota = lax.iota(jnp.int32, 16)
    for ir in range(ncols):
        for j0 in (0, 16):
            v = plsc.load_gather(slab, [iota + j0, jnp.full((16,), ir, jnp.int32)])
            block[ir // 4, pl.ds((ir % 4) * 32 + j0, 16)] = v


@functools.partial(
    pl.kernel,
    mesh=_mesh,
    compiler_params=pltpu.CompilerParams(
        use_tc_tiling_on_sc=True, needs_layout_passes=False
    ),
    out_type=jax.ShapeDtypeStruct((NQUAD, 128), jnp.float32),
    scratch_types=[
        pltpu.VMEM((32, 128), jnp.float32),
        pltpu.VMEM((32, 128), jnp.float32),
        pltpu.VMEM((32, TAIL), jnp.float32),
    ],
)
def _detile_kernel(wt_hbm, scratch_hbm, slab_v, block_v, tail_v):
    wid = lax.axis_index("s") * 2 + lax.axis_index("c")

    def body(c, carry):
        iq = wid + c * NUM_WORKERS
        pltpu.sync_copy(wt_hbm.at[:, pl.ds(iq * 128, 128)], slab_v)
        _transpose_slab(slab_v, block_v, 128)
        pltpu.sync_copy(block_v, scratch_hbm.at[pl.ds(iq * 32, 32)])
        return carry

    count = 244 + jnp.where(wid < NTILE - 244 * NUM_WORKERS, 1, 0)
    lax.fori_loop(0, count, body, 0)

    @pl.when(wid == 0)
    def _tail():
        pltpu.sync_copy(wt_hbm.at[:, pl.ds(NTILE * 128, TAIL)], tail_v)
        _transpose_slab(tail_v, block_v, TAIL)
        pltpu.sync_copy(
            block_v.at[: TAIL // 4], scratch_hbm.at[pl.ds(NTILE * 32, TAIL // 4)]
        )


@functools.partial(
    pl.kernel,
    mesh=_mesh,
    compiler_params=pltpu.CompilerParams(use_tc_tiling_on_sc=False),
    out_type=jax.ShapeDtypeStruct((B_TOTAL, DIM), jnp.float32),
    scratch_types=[
        pltpu.VMEM((BPW,), jnp.int32),
        [pltpu.VMEM((CHUNK, DIM), jnp.float32) for _ in range(NBUF)],
        [pltpu.SemaphoreType.DMA for _ in range(NBUF)],
        [pltpu.SemaphoreType.DMA for _ in range(NBUF)],
    ],
)
def _gather_kernel(idx_hbm, table_hbm, out_hbm, idx_v, bufs, gsems, osems):
    wid = lax.axis_index("s") * 2 + lax.axis_index("c")
    base = wid * BPW
    pltpu.sync_copy(idx_hbm.at[pl.ds(base, BPW)], idx_v)

    def start_gather(g):
        b = g % NBUF
        return pltpu.async_copy(
            table_hbm.at[idx_v.at[pl.ds(g * CHUNK, CHUNK)]], bufs[b], gsems[b]
        )

    def start_ocopy(g):
        b = g % NBUF
        return pltpu.async_copy(
            bufs[b], out_hbm.at[pl.ds(base + g * CHUNK, CHUNK)], osems[b]
        )

    gh = [start_gather(g) for g in range(NBUF)]
    oh = [None] * NBUF
    for g in range(NCHUNK):
        b = g % NBUF
        gh[b].wait()
        oh[b] = start_ocopy(g)
        nxt = g + NBUF
        if nxt < NCHUNK:
            oh[b].wait()
            gh[b] = start_gather(nxt)
    for g in range(NCHUNK - NBUF, NCHUNK):
        oh[g % NBUF].wait()


def kernel(token_ids, weight):
    scratch = _detile_kernel(weight.T)
    table = scratch.reshape(NUM_EMB, DIM)
    flat = token_ids.reshape(-1).astype(jnp.int32)
    out = _gather_kernel(flat, table)
    return out.reshape(token_ids.shape + (DIM,))


# speedup vs baseline: 1.2161x; 1.2161x over previous
import functools

import jax
import jax.numpy as jnp
from jax import lax
from jax.experimental import pallas as pl
from jax.experimental.pallas import tpu as pltpu
from jax.experimental.pallas import tpu_sc as plsc

NUM_EMB = 1000000
DIM = 32
B_TOTAL = 16384 * 26
NUM_WORKERS = 32
BPW = B_TOTAL // NUM_WORKERS
CHUNK = 512
NCHUNK = BPW // CHUNK
NBUF = 4

NTILE = NUM_EMB // 128
TAIL = NUM_EMB - NTILE * 128
NQUAD = NUM_EMB // 4

_mesh = plsc.VectorSubcoreMesh(core_axis_name="c", subcore_axis_name="s")


SLABS_MAIN = 244
SLABS_LEFT = NTILE - SLABS_MAIN * NUM_WORKERS


def _transpose_slab(slab, block, ncols, jvecs):
    for ir in range(ncols):
        irv = jnp.full((16,), ir, jnp.int32)
        for k, jv in enumerate(jvecs):
            v = plsc.load_gather(slab, [jv, irv])
            block[ir // 4, pl.ds((ir % 4) * 32 + k * 16, 16)] = v


@functools.partial(
    pl.kernel,
    mesh=_mesh,
    compiler_params=pltpu.CompilerParams(
        use_tc_tiling_on_sc=True, needs_layout_passes=False
    ),
    out_type=jax.ShapeDtypeStruct((NQUAD, 128), jnp.float32),
    scratch_types=[
        [pltpu.VMEM((32, 128), jnp.float32) for _ in range(2)],
        [pltpu.VMEM((32, 128), jnp.float32) for _ in range(2)],
        pltpu.VMEM((32, TAIL), jnp.float32),
        [pltpu.SemaphoreType.DMA for _ in range(2)],
        [pltpu.SemaphoreType.DMA for _ in range(2)],
    ],
)
def _detile_kernel(wt_hbm, scratch_hbm, slabs, blocks, tail_v, isems, osems):
    wid = lax.axis_index("s") * 2 + lax.axis_index("c")
    base = wid * SLABS_MAIN
    iota = lax.iota(jnp.int32, 16)
    jvecs = (iota, iota + 16)

    def in_copy(s, b):
        return pltpu.make_async_copy(
            wt_hbm.at[:, pl.ds(s * 128, 128)], slabs[b], isems[b]
        )

    def out_copy(s, b):
        return pltpu.make_async_copy(
            blocks[b], scratch_hbm.at[pl.ds(s * 32, 32)], osems[b]
        )

    in_copy(base, 0).start()
    in_copy(base + 1, 1).start()

    def body(g, carry):
        for b in (0, 1):
            c = g * 2 + b
            s = base + c
            in_copy(s, b).wait()
            @pl.when(g > 0)
            def _():
                out_copy(s, b).wait()
            _transpose_slab(slabs[b], blocks[b], 128, jvecs)
            out_copy(s, b).start()
            @pl.when(c + 2 < SLABS_MAIN)
            def _():
                in_copy(s + 2, b).start()
        return carry

    lax.fori_loop(0, SLABS_MAIN // 2, body, 0)
    out_copy(0, 0).wait()
    out_copy(0, 1).wait()

    @pl.when(wid < SLABS_LEFT)
    def _left():
        s = SLABS_MAIN * NUM_WORKERS + wid
        in_copy(s, 0).start()
        in_copy(s, 0).wait()
        _transpose_slab(slabs[0], blocks[0], 128, jvecs)
        out_copy(s, 0).start()
        out_copy(s, 0).wait()

    @pl.when(wid == SLABS_LEFT)
    def _tail():
        pltpu.sync_copy(wt_hbm.at[:, pl.ds(NTILE * 128, TAIL)], tail_v)
        _transpose_slab(tail_v, blocks[0], TAIL, jvecs)
        pltpu.sync_copy(
            blocks[0].at[: TAIL // 4], scratch_hbm.at[pl.ds(NTILE * 32, TAIL // 4)]
        )


@functools.partial(
    pl.kernel,
    mesh=_mesh,
    compiler_params=pltpu.CompilerParams(use_tc_tiling_on_sc=False),
    out_type=jax.ShapeDtypeStruct((B_TOTAL, DIM), jnp.float32),
    scratch_types=[
        pltpu.VMEM((BPW,), jnp.int32),
        [pltpu.VMEM((CHUNK, DIM), jnp.float32) for _ in range(NBUF)],
        [pltpu.SemaphoreType.DMA for _ in range(NBUF)],
        [pltpu.SemaphoreType.DMA for _ in range(NBUF)],
    ],
)
def _gather_kernel(idx_hbm, table_hbm, out_hbm, idx_v, bufs, gsems, osems):
    wid = lax.axis_index("s") * 2 + lax.axis_index("c")
    base = wid * BPW
    pltpu.sync_copy(idx_hbm.at[pl.ds(base, BPW)], idx_v)

    def start_gather(g):
        b = g % NBUF
        return pltpu.async_copy(
            table_hbm.at[idx_v.at[pl.ds(g * CHUNK, CHUNK)]], bufs[b], gsems[b]
        )

    def start_ocopy(g):
        b = g % NBUF
        return pltpu.async_copy(
            bufs[b], out_hbm.at[pl.ds(base + g * CHUNK, CHUNK)], osems[b]
        )

    gh = [start_gather(g) for g in range(NBUF)]
    oh = [None] * NBUF
    for g in range(NCHUNK):
        b = g % NBUF
        gh[b].wait()
        oh[b] = start_ocopy(g)
        nxt = g + NBUF
        if nxt < NCHUNK:
            oh[b].wait()
            gh[b] = start_gather(nxt)
    for g in range(NCHUNK - NBUF, NCHUNK):
        oh[g % NBUF].wait()


def kernel(token_ids, weight):
    scratch = _detile_kernel(weight.T)
    table = scratch.reshape(NUM_EMB, DIM)
    flat = token_ids.reshape(-1).astype(jnp.int32)
    out = _gather_kernel(flat, table)
    return out.reshape(token_ids.shape + (DIM,))


# speedup vs baseline: 1.4887x; 1.2242x over previous
import functools

import jax
import jax.numpy as jnp
from jax import lax
from jax.experimental import pallas as pl
from jax.experimental.pallas import tpu as pltpu
from jax.experimental.pallas import tpu_sc as plsc

NUM_EMB = 1000000
DIM = 32
B_TOTAL = 16384 * 26
NUM_WORKERS = 32
BPW = B_TOTAL // NUM_WORKERS
CHUNK = 512
NCHUNK = BPW // CHUNK
NBUF = 4

NTILE = NUM_EMB // 128
TAIL = NUM_EMB - NTILE * 128
NQUAD = NUM_EMB // 4

_mesh = plsc.VectorSubcoreMesh(core_axis_name="c", subcore_axis_name="s")


SLABS_MAIN = 244
SLABS_LEFT = NTILE - SLABS_MAIN * NUM_WORKERS


def _transpose_slab(slab, block, ncols, rowbase, colbase):
    for j in range(32):
        for c0 in range(0, ncols, 16):
            v = slab[j, pl.ds(c0, 16)]
            plsc.store_scatter(block, [rowbase + (c0 // 4), colbase + j], v)


@functools.partial(
    pl.kernel,
    mesh=_mesh,
    compiler_params=pltpu.CompilerParams(
        use_tc_tiling_on_sc=True, needs_layout_passes=False
    ),
    out_type=jax.ShapeDtypeStruct((NQUAD, 128), jnp.float32),
    scratch_types=[
        [pltpu.VMEM((32, 128), jnp.float32) for _ in range(2)],
        [pltpu.VMEM((32, 128), jnp.float32) for _ in range(2)],
        pltpu.VMEM((32, TAIL), jnp.float32),
        [pltpu.SemaphoreType.DMA for _ in range(2)],
        [pltpu.SemaphoreType.DMA for _ in range(2)],
    ],
)
def _detile_kernel(wt_hbm, scratch_hbm, slabs, blocks, tail_v, isems, osems):
    wid = lax.axis_index("s") * 2 + lax.axis_index("c")
    base = wid * SLABS_MAIN
    iota = lax.iota(jnp.int32, 16)
    rowbase = iota // 4
    colbase = (iota % 4) * 32

    def in_copy(s, b):
        return pltpu.make_async_copy(
            wt_hbm.at[:, pl.ds(s * 128, 128)], slabs[b], isems[b]
        )

    def out_copy(s, b):
        return pltpu.make_async_copy(
            blocks[b], scratch_hbm.at[pl.ds(s * 32, 32)], osems[b]
        )

    in_copy(base, 0).start()
    in_copy(base + 1, 1).start()

    def body(g, carry):
        for b in (0, 1):
            c = g * 2 + b
            s = base + c
            in_copy(s, b).wait()
            @pl.when(g > 0)
            def _():
                out_copy(s, b).wait()
            _transpose_slab(slabs[b], blocks[b], 128, rowbase, colbase)
            out_copy(s, b).start()
            @pl.when(c + 2 < SLABS_MAIN)
            def _():
                in_copy(s + 2, b).start()
        return carry

    lax.fori_loop(0, SLABS_MAIN // 2, body, 0)
    out_copy(0, 0).wait()
    out_copy(0, 1).wait()

    @pl.when(wid < SLABS_LEFT)
    def _left():
        s = SLABS_MAIN * NUM_WORKERS + wid
        in_copy(s, 0).start()
        in_copy(s, 0).wait()
        _transpose_slab(slabs[0], blocks[0], 128, rowbase, colbase)
        out_copy(s, 0).start()
        out_copy(s, 0).wait()

    @pl.when(wid == SLABS_LEFT)
    def _tail():
        pltpu.sync_copy(wt_hbm.at[:, pl.ds(NTILE * 128, TAIL)], tail_v)
        _transpose_slab(tail_v, blocks[0], TAIL, rowbase, colbase)
        pltpu.sync_copy(
            blocks[0].at[: TAIL // 4], scratch_hbm.at[pl.ds(NTILE * 32, TAIL // 4)]
        )


@functools.partial(
    pl.kernel,
    mesh=_mesh,
    compiler_params=pltpu.CompilerParams(use_tc_tiling_on_sc=False),
    out_type=jax.ShapeDtypeStruct((B_TOTAL, DIM), jnp.float32),
    scratch_types=[
        pltpu.VMEM((BPW,), jnp.int32),
        [pltpu.VMEM((CHUNK, DIM), jnp.float32) for _ in range(NBUF)],
        [pltpu.SemaphoreType.DMA for _ in range(NBUF)],
        [pltpu.SemaphoreType.DMA for _ in range(NBUF)],
    ],
)
def _gather_kernel(idx_hbm, table_hbm, out_hbm, idx_v, bufs, gsems, osems):
    wid = lax.axis_index("s") * 2 + lax.axis_index("c")
    base = wid * BPW
    pltpu.sync_copy(idx_hbm.at[pl.ds(base, BPW)], idx_v)

    def start_gather(g):
        b = g % NBUF
        return pltpu.async_copy(
            table_hbm.at[idx_v.at[pl.ds(g * CHUNK, CHUNK)]], bufs[b], gsems[b]
        )

    def start_ocopy(g):
        b = g % NBUF
        return pltpu.async_copy(
            bufs[b], out_hbm.at[pl.ds(base + g * CHUNK, CHUNK)], osems[b]
        )

    gh = [start_gather(g) for g in range(NBUF)]
    oh = [None] * NBUF
    for g in range(NCHUNK):
        b = g % NBUF
        gh[b].wait()
        oh[b] = start_ocopy(g)
        nxt = g + NBUF
        if nxt < NCHUNK:
            oh[b].wait()
            gh[b] = start_gather(nxt)
    for g in range(NCHUNK - NBUF, NCHUNK):
        oh[g % NBUF].wait()


def kernel(token_ids, weight):
    scratch = _detile_kernel(weight.T)
    table = scratch.reshape(NUM_EMB, DIM)
    flat = token_ids.reshape(-1).astype(jnp.int32)
    out = _gather_kernel(flat, table)
    return out.reshape(token_ids.shape + (DIM,))


# speedup vs baseline: 1.8681x; 1.2549x over previous
import functools

import jax
import jax.numpy as jnp
from jax import lax
from jax.experimental import pallas as pl
from jax.experimental.pallas import tpu as pltpu
from jax.experimental.pallas import tpu_sc as plsc

NUM_EMB = 1000000
DIM = 32
B_TOTAL = 16384 * 26
NUM_WORKERS = 32
BPW = B_TOTAL // NUM_WORKERS
CHUNK = 512
NCHUNK = BPW // CHUNK
NBUF = 4

NTILE = NUM_EMB // 128
TAIL = NUM_EMB - NTILE * 128
NQUAD = NUM_EMB // 4

_mesh = plsc.VectorSubcoreMesh(core_axis_name="c", subcore_axis_name="s")


SLABS_MAIN = 244
SLABS_LEFT = NTILE - SLABS_MAIN * NUM_WORKERS


def _transpose_slab(slab, block, ncols, rowbase, colbase):
    @plsc.parallel_loop(0, 32, unroll=8)
    def _rows(j):
        for c0 in range(0, ncols, 16):
            v = slab[j, pl.ds(c0, 16)]
            plsc.store_scatter(block, [rowbase + (c0 // 4), colbase + j], v)


@functools.partial(
    pl.kernel,
    mesh=_mesh,
    compiler_params=pltpu.CompilerParams(
        use_tc_tiling_on_sc=True, needs_layout_passes=False
    ),
    out_type=jax.ShapeDtypeStruct((NQUAD, 128), jnp.float32),
    scratch_types=[
        [pltpu.VMEM((32, 128), jnp.float32) for _ in range(2)],
        [pltpu.VMEM((32, 128), jnp.float32) for _ in range(2)],
        pltpu.VMEM((32, TAIL), jnp.float32),
        [pltpu.SemaphoreType.DMA for _ in range(2)],
        [pltpu.SemaphoreType.DMA for _ in range(2)],
    ],
)
def _detile_kernel(wt_hbm, scratch_hbm, slabs, blocks, tail_v, isems, osems):
    wid = lax.axis_index("s") * 2 + lax.axis_index("c")
    base = wid * SLABS_MAIN
    iota = lax.iota(jnp.int32, 16)
    rowbase = iota // 4
    colbase = (iota % 4) * 32

    def in_copy(s, b):
        return pltpu.make_async_copy(
            wt_hbm.at[:, pl.ds(s * 128, 128)], slabs[b], isems[b]
        )

    def out_copy(s, b):
        return pltpu.make_async_copy(
            blocks[b], scratch_hbm.at[pl.ds(s * 32, 32)], osems[b]
        )

    in_copy(base, 0).start()
    in_copy(base + 1, 1).start()

    def body(g, carry):
        for b in (0, 1):
            c = g * 2 + b
            s = base + c
            in_copy(s, b).wait()
            @pl.when(g > 0)
            def _():
                out_copy(s, b).wait()
            _transpose_slab(slabs[b], blocks[b], 128, rowbase, colbase)
            out_copy(s, b).start()
            @pl.when(c + 2 < SLABS_MAIN)
            def _():
                in_copy(s + 2, b).start()
        return carry

    lax.fori_loop(0, SLABS_MAIN // 2, body, 0)
    out_copy(0, 0).wait()
    out_copy(0, 1).wait()

    @pl.when(wid < SLABS_LEFT)
    def _left():
        s = SLABS_MAIN * NUM_WORKERS + wid
        in_copy(s, 0).start()
        in_copy(s, 0).wait()
        _transpose_slab(slabs[0], blocks[0], 128, rowbase, colbase)
        out_copy(s, 0).start()
        out_copy(s, 0).wait()

    @pl.when(wid == SLABS_LEFT)
    def _tail():
        pltpu.sync_copy(wt_hbm.at[:, pl.ds(NTILE * 128, TAIL)], tail_v)
        _transpose_slab(tail_v, blocks[0], TAIL, rowbase, colbase)
        pltpu.sync_copy(
            blocks[0].at[: TAIL // 4], scratch_hbm.at[pl.ds(NTILE * 32, TAIL // 4)]
        )


@functools.partial(
    pl.kernel,
    mesh=_mesh,
    compiler_params=pltpu.CompilerParams(use_tc_tiling_on_sc=False),
    out_type=jax.ShapeDtypeStruct((B_TOTAL, DIM), jnp.float32),
    scratch_types=[
        pltpu.VMEM((BPW,), jnp.int32),
        [pltpu.VMEM((CHUNK, DIM), jnp.float32) for _ in range(NBUF)],
        [pltpu.SemaphoreType.DMA for _ in range(NBUF)],
        [pltpu.SemaphoreType.DMA for _ in range(NBUF)],
    ],
)
def _gather_kernel(idx_hbm, table_hbm, out_hbm, idx_v, bufs, gsems, osems):
    wid = lax.axis_index("s") * 2 + lax.axis_index("c")
    base = wid * BPW
    pltpu.sync_copy(idx_hbm.at[pl.ds(base, BPW)], idx_v)

    def start_gather(g):
        b = g % NBUF
        return pltpu.async_copy(
            table_hbm.at[idx_v.at[pl.ds(g * CHUNK, CHUNK)]], bufs[b], gsems[b]
        )

    def start_ocopy(g):
        b = g % NBUF
        return pltpu.async_copy(
            bufs[b], out_hbm.at[pl.ds(base + g * CHUNK, CHUNK)], osems[b]
        )

    gh = [start_gather(g) for g in range(NBUF)]
    oh = [None] * NBUF
    for g in range(NCHUNK):
        b = g % NBUF
        gh[b].wait()
        oh[b] = start_ocopy(g)
        nxt = g + NBUF
        if nxt < NCHUNK:
            oh[b].wait()
            gh[b] = start_gather(nxt)
    for g in range(NCHUNK - NBUF, NCHUNK):
        oh[g % NBUF].wait()


def kernel(token_ids, weight):
    scratch = _detile_kernel(weight.T)
    table = scratch.reshape(NUM_EMB, DIM)
    flat = token_ids.reshape(-1).astype(jnp.int32)
    out = _gather_kernel(flat, table)
    return out.reshape(token_ids.shape + (DIM,))


# speedup vs baseline: 2.1482x; 1.1499x over previous
import functools

import jax
import jax.numpy as jnp
from jax import lax
from jax.experimental import pallas as pl
from jax.experimental.pallas import tpu as pltpu
from jax.experimental.pallas import tpu_sc as plsc

NUM_EMB = 1000000
DIM = 32
B_TOTAL = 16384 * 26
NUM_WORKERS = 32
BPW = B_TOTAL // NUM_WORKERS
CHUNK = 512
NCHUNK = BPW // CHUNK
NBUF = 4

NTILE = NUM_EMB // 128
TAIL = NUM_EMB - NTILE * 128
NQUAD = NUM_EMB // 4

_mesh = plsc.VectorSubcoreMesh(core_axis_name="c", subcore_axis_name="s")


SLABS_MAIN = 244
SLABS_LEFT = NTILE - SLABS_MAIN * NUM_WORKERS


def _transpose_slab(slab, block, ncols, rowbase, colbase):
    @plsc.parallel_loop(0, 32, unroll=8)
    def _rows(j):
        for c0 in range(0, ncols, 16):
            v = slab[j, pl.ds(c0, 16)]
            plsc.store_scatter(block, [rowbase + (c0 // 4), colbase + j], v)


@functools.partial(
    pl.kernel,
    mesh=_mesh,
    compiler_params=pltpu.CompilerParams(
        use_tc_tiling_on_sc=True, needs_layout_passes=False
    ),
    out_type=jax.ShapeDtypeStruct((NQUAD, 128), jnp.float32),
    scratch_types=[
        [pltpu.VMEM((32, 128), jnp.float32) for _ in range(2)],
        [pltpu.VMEM((32, 128), jnp.float32) for _ in range(2)],
        pltpu.VMEM((32, TAIL), jnp.float32),
        [pltpu.SemaphoreType.DMA for _ in range(2)],
        [pltpu.SemaphoreType.DMA for _ in range(2)],
    ],
)
def _detile_kernel(wt_hbm, scratch_hbm, slabs, blocks, tail_v, isems, osems):
    wid = lax.axis_index("s") * 2 + lax.axis_index("c")
    base = wid * SLABS_MAIN
    iota = lax.iota(jnp.int32, 16)
    rowbase = iota // 4
    colbase = (iota % 4) * 32

    def in_copy(s, b):
        return pltpu.make_async_copy(
            wt_hbm.at[:, pl.ds(s * 128, 128)], slabs[b], isems[b]
        )

    def out_copy(s, b):
        return pltpu.make_async_copy(
            blocks[b], scratch_hbm.at[pl.ds(s * 32, 32)], osems[b]
        )

    in_copy(base, 0).start()
    in_copy(base + 1, 1).start()

    def body(g, carry):
        for b in (0, 1):
            c = g * 2 + b
            s = base + c
            in_copy(s, b).wait()
            @pl.when(g > 0)
            def _():
                out_copy(s, b).wait()
            _transpose_slab(slabs[b], blocks[b], 128, rowbase, colbase)
            out_copy(s, b).start()
            @pl.when(c + 2 < SLABS_MAIN)
            def _():
                in_copy(s + 2, b).start()
        return carry

    lax.fori_loop(0, SLABS_MAIN // 2, body, 0)
    out_copy(0, 0).wait()
    out_copy(0, 1).wait()

    @pl.when(wid < SLABS_LEFT)
    def _left():
        s = SLABS_MAIN * NUM_WORKERS + wid
        in_copy(s, 0).start()
        in_copy(s, 0).wait()
        _transpose_slab(slabs[0], blocks[0], 128, rowbase, colbase)
        out_copy(s, 0).start()
        out_copy(s, 0).wait()

    @pl.when(wid == SLABS_LEFT)
    def _tail():
        pltpu.sync_copy(wt_hbm.at[:, pl.ds(NTILE * 128, TAIL)], tail_v)
        _transpose_slab(tail_v, blocks[0], TAIL, rowbase, colbase)
        pltpu.sync_copy(
            blocks[0].at[: TAIL // 4], scratch_hbm.at[pl.ds(NTILE * 32, TAIL // 4)]
        )


NT = 26
NB = 16384
NBQ = NB // 128
BLOCKS = NT * NBQ
BLK_PER_W = BLOCKS // NUM_WORKERS


@functools.partial(
    pl.kernel,
    mesh=_mesh,
    compiler_params=pltpu.CompilerParams(
        use_tc_tiling_on_sc=False, needs_layout_passes=False
    ),
    out_type=jax.ShapeDtypeStruct((NT, 4, NBQ, 8, 128), jnp.float32),
    scratch_types=[
        pltpu.VMEM((BPW,), jnp.int32),
        [pltpu.VMEM((128, DIM), jnp.float32) for _ in range(2)],
        [pltpu.VMEM((DIM, 128), jnp.float32) for _ in range(2)],
        [pltpu.SemaphoreType.DMA for _ in range(2)],
        [pltpu.SemaphoreType.DMA for _ in range(2)],
    ],
)
def _gather_kernel(idx_hbm, table_hbm, out_hbm, idx_v, rows, blks, gsems, osems):
    wid = lax.axis_index("s") * 2 + lax.axis_index("c")
    base = wid * BPW
    iota = lax.iota(jnp.int32, 16)
    pltpu.sync_copy(idx_hbm.at[pl.ds(base, BPW)], idx_v)

    def gather(c, b):
        return pltpu.make_async_copy(
            table_hbm.at[idx_v.at[pl.ds(c * 128, 128)]], rows[b], gsems[b]
        )

    def out_copies(c, b):
        blk_id = wid * BLK_PER_W + c
        t = blk_id // NBQ
        bq = blk_id - t * NBQ
        return [
            pltpu.make_async_copy(
                blks[b].at[pl.ds(jq * 8, 8)], out_hbm.at[t, jq, bq], osems[b]
            )
            for jq in range(4)
        ]

    def transpose(b):
        @plsc.parallel_loop(0, DIM, unroll=8)
        def _rows(j):
            jv = jnp.full((16,), j, jnp.int32)
            for c0 in range(0, 128, 16):
                v = plsc.load_gather(rows[b], [iota + c0, jv])
                blks[b][j, pl.ds(c0, 16)] = v

    gather(0, 0).start()
    gather(1, 1).start()

    def body(g, carry):
        for b in (0, 1):
            c = g * 2 + b
            gather(c, b).wait()
            @pl.when(g > 0)
            def _():
                for h in out_copies(c, b):
                    h.wait()
            transpose(b)
            for h in out_copies(c, b):
                h.start()
            @pl.when(c + 2 < BLK_PER_W)
            def _():
                gather(c + 2, b).start()
        return carry

    lax.fori_loop(0, BLK_PER_W // 2, body, 0)
    for b in (0, 1):
        for h in out_copies(0, b):
            h.wait()


def kernel(token_ids, weight):
    scratch = _detile_kernel(weight.T)
    table = scratch.reshape(NUM_EMB, DIM)
    flat = token_ids.T.reshape(-1).astype(jnp.int32)
    out4 = _gather_kernel(flat, table)
    out = out4.transpose(2, 4, 0, 1, 3).reshape(NB, NT, DIM)
    return out


# speedup vs baseline: 3.6825x; 1.7142x over previous
import functools

import jax
import jax.numpy as jnp
from jax import lax
from jax.experimental import pallas as pl
from jax.experimental.pallas import tpu as pltpu
from jax.experimental.pallas import tpu_sc as plsc

NUM_EMB = 1000000
DIM = 32
B_TOTAL = 16384 * 26
NUM_WORKERS = 32
BPW = B_TOTAL // NUM_WORKERS
CHUNK = 512
NCHUNK = BPW // CHUNK
NBUF = 4

NTILE = NUM_EMB // 128
TAIL = NUM_EMB - NTILE * 128
NQUAD = NUM_EMB // 4

_mesh = plsc.VectorSubcoreMesh(core_axis_name="c", subcore_axis_name="s")


SLABS_MAIN = 244
SLABS_LEFT = NTILE - SLABS_MAIN * NUM_WORKERS


PITCH = 136


def _transpose_slab(slab, block, ncols, pitchbase):
    @plsc.parallel_loop(0, 32, unroll=8)
    def _rows(j):
        for c0 in range(0, ncols, 16):
            v = slab[j, pl.ds(c0, 16)]
            plsc.store_scatter(block, [pitchbase + ((c0 // 4) * PITCH + j)], v)


@functools.partial(
    pl.kernel,
    mesh=_mesh,
    compiler_params=pltpu.CompilerParams(
        use_tc_tiling_on_sc=True, needs_layout_passes=False
    ),
    out_type=jax.ShapeDtypeStruct((NUM_EMB * DIM,), jnp.float32),
    scratch_types=[
        [pltpu.VMEM((32, 128), jnp.float32) for _ in range(2)],
        [pltpu.VMEM((32 * PITCH,), jnp.float32) for _ in range(2)],
        pltpu.VMEM((32, TAIL), jnp.float32),
        [pltpu.SemaphoreType.DMA for _ in range(2)],
        [pltpu.SemaphoreType.DMA for _ in range(2)],
    ],
)
def _detile_kernel(wt_hbm, scratch_hbm, slabs, blocks, tail_v, isems, osems):
    wid = lax.axis_index("s") * 2 + lax.axis_index("c")
    base = wid * SLABS_MAIN
    iota = lax.iota(jnp.int32, 16)
    pitchbase = (iota // 4) * PITCH + (iota % 4) * 32

    def in_copy(s, b):
        return pltpu.make_async_copy(
            wt_hbm.at[:, pl.ds(s * 128, 128)], slabs[b], isems[b]
        )

    def out_rows(s, b, nq=32):
        return [
            pltpu.make_async_copy(
                blocks[b].at[pl.ds(q * PITCH, 128)],
                scratch_hbm.at[pl.ds((s * 32 + q) * 128, 128)],
                osems[b],
            )
            for q in range(nq)
        ]

    in_copy(base, 0).start()
    in_copy(base + 1, 1).start()

    def body(g, carry):
        for b in (0, 1):
            c = g * 2 + b
            s = base + c
            in_copy(s, b).wait()
            @pl.when(g > 0)
            def _():
                for h in out_rows(s, b):
                    h.wait()
            _transpose_slab(slabs[b], blocks[b], 128, pitchbase)
            for h in out_rows(s, b):
                h.start()
            @pl.when(c + 2 < SLABS_MAIN)
            def _():
                in_copy(s + 2, b).start()
        return carry

    lax.fori_loop(0, SLABS_MAIN // 2, body, 0)
    for b in (0, 1):
        for h in out_rows(0, b):
            h.wait()

    @pl.when(wid < SLABS_LEFT)
    def _left():
        s = SLABS_MAIN * NUM_WORKERS + wid
        in_copy(s, 0).start()
        in_copy(s, 0).wait()
        _transpose_slab(slabs[0], blocks[0], 128, pitchbase)
        for h in out_rows(s, 0):
            h.start()
        for h in out_rows(s, 0):
            h.wait()

    @pl.when(wid == SLABS_LEFT)
    def _tail():
        pltpu.sync_copy(wt_hbm.at[:, pl.ds(NTILE * 128, TAIL)], tail_v)
        _transpose_slab(tail_v, blocks[0], TAIL, pitchbase)
        for q in range(TAIL // 4):
            pltpu.sync_copy(
                blocks[0].at[pl.ds(q * PITCH, 128)],
                scratch_hbm.at[pl.ds((NTILE * 32 + q) * 128, 128)],
            )


NT = 26
NB = 16384
NBQ = NB // 128
BLOCKS = NT * NBQ
BLK_PER_W = BLOCKS // NUM_WORKERS


@functools.partial(
    pl.kernel,
    mesh=_mesh,
    compiler_params=pltpu.CompilerParams(
        use_tc_tiling_on_sc=False, needs_layout_passes=False
    ),
    out_type=jax.ShapeDtypeStruct((NT, 4, NBQ, 8, 128), jnp.float32),
    scratch_types=[
        pltpu.VMEM((BPW,), jnp.int32),
        [pltpu.VMEM((128, DIM), jnp.float32) for _ in range(2)],
        [pltpu.VMEM((DIM, 128), jnp.float32) for _ in range(2)],
        [pltpu.SemaphoreType.DMA for _ in range(2)],
        [pltpu.SemaphoreType.DMA for _ in range(2)],
    ],
)
def _gather_kernel(idx_hbm, table_hbm, out_hbm, idx_v, rows, blks, gsems, osems):
    wid = lax.axis_index("s") * 2 + lax.axis_index("c")
    base = wid * BPW
    iota = lax.iota(jnp.int32, 16)
    pltpu.sync_copy(idx_hbm.at[pl.ds(base, BPW)], idx_v)

    def gather(c, b):
        return pltpu.make_async_copy(
            table_hbm.at[idx_v.at[pl.ds(c * 128, 128)]], rows[b], gsems[b]
        )

    def out_copies(c, b):
        blk_id = wid * BLK_PER_W + c
        t = blk_id // NBQ
        bq = blk_id - t * NBQ
        return [
            pltpu.make_async_copy(
                blks[b].at[pl.ds(jq * 8, 8)], out_hbm.at[t, jq, bq], osems[b]
            )
            for jq in range(4)
        ]

    def transpose(b):
        @plsc.parallel_loop(0, DIM, unroll=8)
        def _rows(j):
            jv = jnp.full((16,), j, jnp.int32)
            for c0 in range(0, 128, 16):
                v = plsc.load_gather(rows[b], [iota + c0, jv])
                blks[b][j, pl.ds(c0, 16)] = v

    gather(0, 0).start()
    gather(1, 1).start()

    def body(g, carry):
        for b in (0, 1):
            c = g * 2 + b
            gather(c, b).wait()
            @pl.when(g > 0)
            def _():
                for h in out_copies(c, b):
                    h.wait()
            transpose(b)
            for h in out_copies(c, b):
                h.start()
            @pl.when(c + 2 < BLK_PER_W)
            def _():
                gather(c + 2, b).start()
        return carry

    lax.fori_loop(0, BLK_PER_W // 2, body, 0)
    for b in (0, 1):
        for h in out_copies(0, b):
            h.wait()


def kernel(token_ids, weight):
    scratch = _detile_kernel(weight.T)
    table = scratch.reshape(NUM_EMB, DIM)
    flat = token_ids.T.reshape(-1).astype(jnp.int32)
    out4 = _gather_kernel(flat, table)
    out = out4.transpose(2, 4, 0, 1, 3).reshape(NB, NT, DIM)
    return out


# speedup vs baseline: 5.2956x; 1.4380x over previous
import functools

import jax
import jax.numpy as jnp
from jax import lax
from jax.experimental import pallas as pl
from jax.experimental.pallas import tpu as pltpu
from jax.experimental.pallas import tpu_sc as plsc

NUM_EMB = 1000000
DIM = 32
B_TOTAL = 16384 * 26
NUM_WORKERS = 32
BPW = B_TOTAL // NUM_WORKERS
CHUNK = 512
NCHUNK = BPW // CHUNK
NBUF = 4

NTILE = NUM_EMB // 128
TAIL = NUM_EMB - NTILE * 128
NQUAD = NUM_EMB // 4

_mesh = plsc.VectorSubcoreMesh(core_axis_name="c", subcore_axis_name="s")


SLABS_MAIN = 244
SLABS_LEFT = NTILE - SLABS_MAIN * NUM_WORKERS


PITCH = 136


def _transpose_slab(slab, block, ncols, pitchbase):
    @plsc.parallel_loop(0, 32, unroll=8)
    def _rows(j):
        for c0 in range(0, ncols, 16):
            v = slab[j, pl.ds(c0, 16)]
            plsc.store_scatter(block, [pitchbase + ((c0 // 4) * PITCH + j)], v)


@functools.partial(
    pl.kernel,
    mesh=_mesh,
    compiler_params=pltpu.CompilerParams(
        use_tc_tiling_on_sc=True, needs_layout_passes=False
    ),
    out_type=jax.ShapeDtypeStruct((NUM_EMB * DIM,), jnp.float32),
    scratch_types=[
        [pltpu.VMEM((32, 128), jnp.float32) for _ in range(2)],
        [pltpu.VMEM((32 * PITCH,), jnp.float32) for _ in range(2)],
        pltpu.VMEM((32, TAIL), jnp.float32),
        [pltpu.SemaphoreType.DMA for _ in range(2)],
        [pltpu.SemaphoreType.DMA for _ in range(2)],
    ],
)
def _detile_kernel(wt_hbm, scratch_hbm, slabs, blocks, tail_v, isems, osems):
    wid = lax.axis_index("s") * 2 + lax.axis_index("c")
    base = wid * SLABS_MAIN
    iota = lax.iota(jnp.int32, 16)
    pitchbase = (iota // 4) * PITCH + (iota % 4) * 32

    def in_copy(s, b):
        return pltpu.make_async_copy(
            wt_hbm.at[:, pl.ds(s * 128, 128)], slabs[b], isems[b]
        )

    def out_rows(s, b, nq=32):
        return [
            pltpu.make_async_copy(
                blocks[b].at[pl.ds(q * PITCH, 128)],
                scratch_hbm.at[pl.ds((s * 32 + q) * 128, 128)],
                osems[b],
            )
            for q in range(nq)
        ]

    in_copy(base, 0).start()
    in_copy(base + 1, 1).start()

    def body(g, carry):
        for b in (0, 1):
            c = g * 2 + b
            s = base + c
            in_copy(s, b).wait()
            @pl.when(g > 0)
            def _():
                for h in out_rows(s, b):
                    h.wait()
            _transpose_slab(slabs[b], blocks[b], 128, pitchbase)
            for h in out_rows(s, b):
                h.start()
            @pl.when(c + 2 < SLABS_MAIN)
            def _():
                in_copy(s + 2, b).start()
        return carry

    lax.fori_loop(0, SLABS_MAIN // 2, body, 0)
    for b in (0, 1):
        for h in out_rows(0, b):
            h.wait()

    @pl.when(wid < SLABS_LEFT)
    def _left():
        s = SLABS_MAIN * NUM_WORKERS + wid
        in_copy(s, 0).start()
        in_copy(s, 0).wait()
        _transpose_slab(slabs[0], blocks[0], 128, pitchbase)
        for h in out_rows(s, 0):
            h.start()
        for h in out_rows(s, 0):
            h.wait()

    @pl.when(wid == SLABS_LEFT)
    def _tail():
        pltpu.sync_copy(wt_hbm.at[:, pl.ds(NTILE * 128, TAIL)], tail_v)
        _transpose_slab(tail_v, blocks[0], TAIL, pitchbase)
        for q in range(TAIL // 4):
            pltpu.sync_copy(
                blocks[0].at[pl.ds(q * PITCH, 128)],
                scratch_hbm.at[pl.ds((NTILE * 32 + q) * 128, 128)],
            )


NT = 26
NB = 16384
NBQ = NB // 128
BLOCKS = NT * NBQ
BLK_PER_W = BLOCKS // NUM_WORKERS


@functools.partial(
    pl.kernel,
    mesh=_mesh,
    compiler_params=pltpu.CompilerParams(
        use_tc_tiling_on_sc=False, needs_layout_passes=False
    ),
    out_type=jax.ShapeDtypeStruct((NT, 4, NBQ, 8, 128), jnp.float32),
    scratch_types=[
        pltpu.VMEM((BPW,), jnp.int32),
        [pltpu.VMEM((128, DIM), jnp.float32) for _ in range(2)],
        [pltpu.VMEM((DIM * PITCH,), jnp.float32) for _ in range(2)],
        [pltpu.SemaphoreType.DMA for _ in range(2)],
        [pltpu.SemaphoreType.DMA for _ in range(2)],
    ],
)
def _gather_kernel(idx_hbm, table_hbm, out_hbm, idx_v, rows, blks, gsems, osems):
    wid = lax.axis_index("s") * 2 + lax.axis_index("c")
    base = wid * BPW
    iota = lax.iota(jnp.int32, 16)
    pltpu.sync_copy(idx_hbm.at[pl.ds(base, BPW)], idx_v)

    def gather(c, b):
        return pltpu.make_async_copy(
            table_hbm.at[idx_v.at[pl.ds(c * 128, 128)]], rows[b], gsems[b]
        )

    def out_copies(c, b):
        blk_id = wid * BLK_PER_W + c
        t = blk_id // NBQ
        bq = blk_id - t * NBQ
        return [
            pltpu.make_async_copy(
                blks[b].at[pl.ds(j * PITCH, 128)],
                out_hbm.at[t, j // 8, bq, j % 8],
                osems[b],
            )
            for j in range(DIM)
        ]

    def transpose(b):
        @plsc.parallel_loop(0, 128, unroll=8)
        def _rows(br):
            for j0 in (0, 16):
                v = rows[b][br, pl.ds(j0, 16)]
                plsc.store_scatter(blks[b], [(iota + j0) * PITCH + br], v)

    gather(0, 0).start()
    gather(1, 1).start()

    def body(g, carry):
        for b in (0, 1):
            c = g * 2 + b
            gather(c, b).wait()
            @pl.when(g > 0)
            def _():
                for h in out_copies(c, b):
                    h.wait()
            transpose(b)
            for h in out_copies(c, b):
                h.start()
            @pl.when(c + 2 < BLK_PER_W)
            def _():
                gather(c + 2, b).start()
        return carry

    lax.fori_loop(0, BLK_PER_W // 2, body, 0)
    for b in (0, 1):
        for h in out_copies(0, b):
            h.wait()


def kernel(token_ids, weight):
    scratch = _detile_kernel(weight.T)
    table = scratch.reshape(NUM_EMB, DIM)
    flat = token_ids.T.reshape(-1).astype(jnp.int32)
    out4 = _gather_kernel(flat, table)
    out = out4.transpose(2, 4, 0, 1, 3).reshape(NB, NT, DIM)
    return out


# speedup vs baseline: 5.5537x; 1.0487x over previous
import functools

import jax
import jax.numpy as jnp
from jax import lax
from jax.experimental import pallas as pl
from jax.experimental.pallas import tpu as pltpu
from jax.experimental.pallas import tpu_sc as plsc

NUM_EMB = 1000000
DIM = 32
B_TOTAL = 16384 * 26
NUM_WORKERS = 32
BPW = B_TOTAL // NUM_WORKERS
CHUNK = 512
NCHUNK = BPW // CHUNK
NBUF = 4

NTILE = NUM_EMB // 128
TAIL = NUM_EMB - NTILE * 128
NQUAD = NUM_EMB // 4

_mesh = plsc.VectorSubcoreMesh(core_axis_name="c", subcore_axis_name="s")


SLABS_MAIN = 244
SLABS_LEFT = NTILE - SLABS_MAIN * NUM_WORKERS


PITCH = 264


def _transpose_slab(slab, block, ncols, pitchbase):
    @plsc.parallel_loop(0, 32, unroll=8)
    def _rows(j):
        for c0 in range(0, ncols, 16):
            v = slab[j, pl.ds(c0, 16)]
            plsc.store_scatter(block, [pitchbase + ((c0 // 8) * PITCH + j)], v)


@functools.partial(
    pl.kernel,
    mesh=_mesh,
    compiler_params=pltpu.CompilerParams(
        use_tc_tiling_on_sc=True, needs_layout_passes=False
    ),
    out_type=jax.ShapeDtypeStruct((NUM_EMB * DIM,), jnp.float32),
    scratch_types=[
        [pltpu.VMEM((32, 128), jnp.float32) for _ in range(2)],
        [pltpu.VMEM((16 * PITCH,), jnp.float32) for _ in range(2)],
        pltpu.VMEM((32, TAIL), jnp.float32),
        [pltpu.SemaphoreType.DMA for _ in range(2)],
        [pltpu.SemaphoreType.DMA for _ in range(2)],
    ],
)
def _detile_kernel(wt_hbm, scratch_hbm, slabs, blocks, tail_v, isems, osems):
    wid = lax.axis_index("s") * 2 + lax.axis_index("c")
    base = wid * SLABS_MAIN
    iota = lax.iota(jnp.int32, 16)
    pitchbase = ((iota // 4) // 2) * PITCH + ((iota // 4) % 2) * 128 + (iota % 4) * 32

    def in_copy(s, b):
        return pltpu.make_async_copy(
            wt_hbm.at[:, pl.ds(s * 128, 128)], slabs[b], isems[b]
        )

    def out_rows(s, b, nq=32):
        return [
            pltpu.make_async_copy(
                blocks[b].at[pl.ds(q2 * PITCH, 256)],
                scratch_hbm.at[pl.ds((s * 32 + q2 * 2) * 128, 256)],
                osems[b],
            )
            for q2 in range(nq // 2)
        ]

    in_copy(base, 0).start()
    in_copy(base + 1, 1).start()

    def body(g, carry):
        for b in (0, 1):
            c = g * 2 + b
            s = base + c
            in_copy(s, b).wait()
            @pl.when(g > 0)
            def _():
                for h in out_rows(s, b):
                    h.wait()
            _transpose_slab(slabs[b], blocks[b], 128, pitchbase)
            for h in out_rows(s, b):
                h.start()
            @pl.when(c + 2 < SLABS_MAIN)
            def _():
                in_copy(s + 2, b).start()
        return carry

    lax.fori_loop(0, SLABS_MAIN // 2, body, 0)
    for b in (0, 1):
        for h in out_rows(0, b):
            h.wait()

    @pl.when(wid < SLABS_LEFT)
    def _left():
        s = SLABS_MAIN * NUM_WORKERS + wid
        in_copy(s, 0).start()
        in_copy(s, 0).wait()
        _transpose_slab(slabs[0], blocks[0], 128, pitchbase)
        for h in out_rows(s, 0):
            h.start()
        for h in out_rows(s, 0):
            h.wait()

    @pl.when(wid == SLABS_LEFT)
    def _tail():
        pltpu.sync_copy(wt_hbm.at[:, pl.ds(NTILE * 128, TAIL)], tail_v)
        _transpose_slab(tail_v, blocks[0], TAIL, pitchbase)
        for q2 in range(TAIL // 8):
            pltpu.sync_copy(
                blocks[0].at[pl.ds(q2 * PITCH, 256)],
                scratch_hbm.at[pl.ds((NTILE * 32 + q2 * 2) * 128, 256)],
            )


NT = 26
NB = 16384
NBQ = NB // 128
BLOCKS = NT * NBQ
BLK_PER_W = BLOCKS // NUM_WORKERS


@functools.partial(
    pl.kernel,
    mesh=_mesh,
    compiler_params=pltpu.CompilerParams(
        use_tc_tiling_on_sc=False, needs_layout_passes=False
    ),
    out_type=jax.ShapeDtypeStruct((NT, 4, NBQ, 1024), jnp.float32),
    scratch_types=[
        pltpu.VMEM((BPW,), jnp.int32),
        [pltpu.VMEM((128, DIM), jnp.float32) for _ in range(2)],
        [pltpu.VMEM((16 * PITCH,), jnp.float32) for _ in range(2)],
        [pltpu.SemaphoreType.DMA for _ in range(2)],
        [pltpu.SemaphoreType.DMA for _ in range(2)],
    ],
)
def _gather_kernel(idx_hbm, table_hbm, out_hbm, idx_v, rows, blks, gsems, osems):
    wid = lax.axis_index("s") * 2 + lax.axis_index("c")
    base = wid * BPW
    iota = lax.iota(jnp.int32, 16)
    pltpu.sync_copy(idx_hbm.at[pl.ds(base, BPW)], idx_v)

    def gather(c, b):
        return pltpu.make_async_copy(
            table_hbm.at[idx_v.at[pl.ds(c * 128, 128)]], rows[b], gsems[b]
        )

    def out_copies(c, b):
        blk_id = wid * BLK_PER_W + c
        t = blk_id // NBQ
        bq = blk_id - t * NBQ
        return [
            pltpu.make_async_copy(
                blks[b].at[pl.ds(p * PITCH, 256)],
                out_hbm.at[t, p // 4, bq, pl.ds((p % 4) * 256, 256)],
                osems[b],
            )
            for p in range(DIM // 2)
        ]

    jbase = (iota // 2) * PITCH + (iota % 2) * 128

    def transpose(b):
        @plsc.parallel_loop(0, 128, unroll=8)
        def _rows(br):
            for j0 in (0, 16):
                v = rows[b][br, pl.ds(j0, 16)]
                plsc.store_scatter(blks[b], [jbase + ((j0 // 2) * PITCH + br)], v)

    gather(0, 0).start()
    gather(1, 1).start()

    def body(g, carry):
        for b in (0, 1):
            c = g * 2 + b
            gather(c, b).wait()
            @pl.when(g > 0)
            def _():
                for h in out_copies(c, b):
                    h.wait()
            transpose(b)
            for h in out_copies(c, b):
                h.start()
            @pl.when(c + 2 < BLK_PER_W)
            def _():
                gather(c + 2, b).start()
        return carry

    lax.fori_loop(0, BLK_PER_W // 2, body, 0)
    for b in (0, 1):
        for h in out_copies(0, b):
            h.wait()


def kernel(token_ids, weight):
    scratch = _detile_kernel(weight.T)
    table = scratch.reshape(NUM_EMB, DIM)
    flat = token_ids.T.reshape(-1).astype(jnp.int32)
    out4 = _gather_kernel(flat, table)
    out5 = out4.reshape(NT, 4, NBQ, 8, 128)
    out = out5.transpose(2, 4, 0, 1, 3).reshape(NB, NT, DIM)
    return out
